# Initial kernel scaffold; baseline (speedup 1.0000x reference)
import jax
import jax.numpy as jnp
from jax.experimental import pallas as pl


def kernel(inputs, table):
    raise NotImplementedError("write your pallas kernel here")


# speedup vs baseline: 1.1866x; 1.1866x over previous
import functools

import jax
import jax.numpy as jnp
from jax import lax
from jax.experimental import pallas as pl
from jax.experimental.pallas import tpu as pltpu
from jax.experimental.pallas import tpu_sc as plsc

VOCAB = 1000000
EMBED_DIM = 32
BATCH = 16384
HIST = 50

NC = 2
NS = 16
NW = NC * NS
B = BATCH * HIST
BPW = B // NW
CHUNK = 128
NCHUNK = BPW // CHUNK


def _emb_body(table_hbm, idx_hbm, out_hbm, idx_v, rows_v, sem):
    wid = lax.axis_index("s") * NC + lax.axis_index("c")
    pltpu.sync_copy(idx_hbm.at[wid], idx_v)

    def step(c, carry):
        pltpu.async_copy(table_hbm.at[idx_v.at[c]], rows_v, sem).wait()
        pltpu.sync_copy(rows_v, out_hbm.at[wid, c])
        return carry

    lax.fori_loop(0, NCHUNK, step, 0)


_emb_call = functools.partial(
    pl.kernel,
    mesh=plsc.VectorSubcoreMesh(core_axis_name="c", subcore_axis_name="s"),
    out_type=jax.ShapeDtypeStruct((NW, NCHUNK, CHUNK, EMBED_DIM), jnp.float32),
    compiler_params=pltpu.CompilerParams(use_tc_tiling_on_sc=False),
    scratch_types=[
        pltpu.VMEM((NCHUNK, CHUNK), jnp.int32),
        pltpu.VMEM((CHUNK, EMBED_DIM), jnp.float32),
        pltpu.SemaphoreType.DMA,
    ],
)(_emb_body)


@jax.jit
def kernel(inputs, table):
    idx = inputs.astype(jnp.int32).reshape(NW, NCHUNK, CHUNK)
    out = _emb_call(table, idx)
    return out.reshape(BATCH, HIST, EMBED_DIM)


# speedup vs baseline: 1.2882x; 1.0857x over previous
import functools

import jax
import jax.numpy as jnp
from jax import lax
from jax.experimental import pallas as pl
from jax.experimental.pallas import tpu as pltpu
from jax.experimental.pallas import tpu_sc as plsc

VOCAB = 1000000
EMBED_DIM = 32
BATCH = 16384
HIST = 50

NC = 2
NS = 16
NW = NC * NS
B = BATCH * HIST
BPW = B // NW
CHUNK = 128
NCHUNK = BPW // CHUNK
GROUP = 10
NGROUP = NCHUNK // GROUP
NP = NGROUP // 2


def _emb_body(table_hbm, idx_hbm, out_hbm, idx_v, buf_a, buf_b,
              sga, sgb, ssa, ssb):
    wid = lax.axis_index("s") * NC + lax.axis_index("c")
    pltpu.sync_copy(idx_hbm.at[wid], idx_v)

    def fire_group(grp, buf, sem):
        for j in range(GROUP):
            pltpu.async_copy(
                table_hbm.at[idx_v.at[grp * GROUP + j]],
                buf.at[pl.ds(j * CHUNK, CHUNK)],
                sem,
            )

    def drain_group(grp, buf, sem):
        for j in range(GROUP):
            pltpu.make_async_copy(
                table_hbm.at[idx_v.at[grp * GROUP + j]],
                buf.at[pl.ds(j * CHUNK, CHUNK)],
                sem,
            ).wait()

    def fire_scatter(grp, buf, sem):
        pltpu.async_copy(buf, out_hbm.at[wid, grp], sem)

    def wait_scatter(grp, buf, sem):
        pltpu.make_async_copy(buf, out_hbm.at[wid, grp], sem).wait()

    fire_group(0, buf_a, sga)

    def body(p, carry):
        ga = 2 * p
        gb = 2 * p + 1

        @pl.when(p > 0)
        def _():
            wait_scatter(ga - 1, buf_b, ssb)
        fire_group(gb, buf_b, sgb)

        drain_group(ga, buf_a, sga)
        fire_scatter(ga, buf_a, ssa)

        @pl.when(p + 1 < NP)
        def _():
            wait_scatter(ga, buf_a, ssa)
            fire_group(ga + 2, buf_a, sga)

        drain_group(gb, buf_b, sgb)
        fire_scatter(gb, buf_b, ssb)
        return carry

    lax.fori_loop(0, NP, body, 0)
    wait_scatter(NGROUP - 2, buf_a, ssa)
    wait_scatter(NGROUP - 1, buf_b, ssb)


_emb_call = functools.partial(
    pl.kernel,
    mesh=plsc.VectorSubcoreMesh(core_axis_name="c", subcore_axis_name="s"),
    out_type=jax.ShapeDtypeStruct(
        (NW, NGROUP, GROUP * CHUNK, EMBED_DIM), jnp.float32),
    compiler_params=pltpu.CompilerParams(use_tc_tiling_on_sc=False),
    scratch_types=[
        pltpu.VMEM((NCHUNK, CHUNK), jnp.int32),
        pltpu.VMEM((GROUP * CHUNK, EMBED_DIM), jnp.float32),
        pltpu.VMEM((GROUP * CHUNK, EMBED_DIM), jnp.float32),
        pltpu.SemaphoreType.DMA,
        pltpu.SemaphoreType.DMA,
        pltpu.SemaphoreType.DMA,
        pltpu.SemaphoreType.DMA,
    ],
)(_emb_body)


@jax.jit
def kernel(inputs, table):
    idx = inputs.astype(jnp.int32).reshape(NW, NCHUNK, CHUNK)
    out = _emb_call(table, idx)
    return out.reshape(BATCH, HIST, EMBED_DIM)
